# baseline (speedup 1.0000x reference)
import functools

import jax
import jax.numpy as jnp
from jax import lax
from jax.experimental import pallas as pl
from jax.experimental.pallas import tpu as pltpu
from jax.experimental.pallas import tpu_sc as plsc

_B = 16384
_F = 32
_G = 3
_NBUF = 5


@functools.cache
def _build(num_rows):
    info = plsc.get_sparse_core_info()
    nc, ns, nl = info.num_cores, info.num_subcores, info.num_lanes
    nw = nc * ns
    bpw = _B // nw
    ngrp = (bpw + _G - 1) // _G
    ngrp = ((ngrp + _NBUF - 1) // _NBUF) * _NBUF
    pad = ngrp * _G + nl
    max_c = (num_rows + 127) // 128 - 1
    mesh = plsc.VectorSubcoreMesh(core_axis_name="c", subcore_axis_name="s")

    @functools.partial(
        pl.kernel,
        mesh=mesh,
        out_type=jax.ShapeDtypeStruct((_B,), jnp.float32),
        compiler_params=pltpu.CompilerParams(needs_layout_passes=False),
        scratch_types=[
            pltpu.VMEM((pad,), jnp.int32),
            pltpu.VMEM((pad,), jnp.int32),
            pltpu.VMEM((2 * _G, _F, 128), jnp.float32),
            pltpu.VMEM((2 * _G, _F, 128), jnp.float32),
            pltpu.VMEM((2 * _G, _F, 128), jnp.float32),
            pltpu.VMEM((2 * _G, _F, 128), jnp.float32),
            pltpu.VMEM((2 * _G, _F, 128), jnp.float32),
            pltpu.VMEM((pad,), jnp.float32),
            pltpu.SemaphoreType.DMA,
            pltpu.SemaphoreType.DMA,
            pltpu.SemaphoreType.DMA,
            pltpu.SemaphoreType.DMA,
            pltpu.SemaphoreType.DMA,
            pltpu.SemaphoreType.DMA,
            pltpu.SemaphoreType.DMA,
            pltpu.SemaphoreType.DMA,
            pltpu.SemaphoreType.DMA,
            pltpu.SemaphoreType.DMA,
        ],
    )
    def sc_dot(ut_h, it_h, uidx_h, iidx_h, out_h,
               uidx_v, iidx_v, b0, b1, b2, b3, b4, out_v,
               su0, si0, su1, si1, su2, si2, su3, si3, su4, si4):
        wid = lax.axis_index("s") * nc + lax.axis_index("c")
        base = wid * bpw
        pltpu.sync_copy(uidx_h.at[pl.ds(base, bpw)], uidx_v.at[pl.ds(0, bpw)])
        pltpu.sync_copy(iidx_h.at[pl.ds(base, bpw)], iidx_v.at[pl.ds(0, bpw)])

        bufs = (b0, b1, b2, b3, b4)
        sems = ((su0, si0), (su1, si1), (su2, si2), (su3, si3), (su4, si4))
        lanes = lax.iota(jnp.int32, nl)
        blk_u = jnp.minimum(2 * lanes, 2 * _G - 2)
        blk_i = jnp.minimum(2 * lanes + 1, 2 * _G - 1)

        def fire(g, b):
            buf, (sem_u, sem_i) = bufs[b], sems[b]
            iv_u = uidx_v[pl.ds(g * _G, nl)]
            iv_i = iidx_v[pl.ds(g * _G, nl)]
            for j in range(_G):
                c_u = jnp.clip(iv_u[j] >> 7, 0, max_c)
                c_i = jnp.clip(iv_i[j] >> 7, 0, max_c)
                off_u = pl.multiple_of(c_u * 128, 128)
                off_i = pl.multiple_of(c_i * 128, 128)
                pltpu.async_copy(ut_h.at[pl.ds(0, _F), pl.ds(off_u, 128)],
                                 buf.at[2 * j], sem_u)
                pltpu.async_copy(it_h.at[pl.ds(0, _F), pl.ds(off_i, 128)],
                                 buf.at[2 * j + 1], sem_i)

        def drain_compute(g, b):
            buf, (sem_u, sem_i) = bufs[b], sems[b]
            for j in range(_G):
                pltpu.make_async_copy(ut_h.at[pl.ds(0, _F), pl.ds(0, 128)],
                                      buf.at[2 * j], sem_u).wait()
                pltpu.make_async_copy(it_h.at[pl.ds(0, _F), pl.ds(0, 128)],
                                      buf.at[2 * j + 1], sem_i).wait()
            iv_u = uidx_v[pl.ds(g * _G, nl)]
            iv_i = iidx_v[pl.ds(g * _G, nl)]
            q_u = jnp.bitwise_and(iv_u, 127)
            q_i = jnp.bitwise_and(iv_i, 127)
            acc = jnp.zeros((nl,), jnp.float32)
            for f in range(_F):
                fv = jnp.full((nl,), f, jnp.int32)
                gu = plsc.load_gather(buf, [blk_u, fv, q_u])
                gi = plsc.load_gather(buf, [blk_i, fv, q_i])
                acc = acc + gu * gi
            out_v[pl.ds(g * _G, nl)] = acc

        for b in range(_NBUF - 1):
            fire(b, b)

        def body(p, carry):
            g0 = _NBUF * p
            for r in range(_NBUF):
                g = g0 + r

                @pl.when(g + _NBUF - 1 < ngrp)
                def _(g=g, r=r):
                    fire(g + _NBUF - 1, (r + _NBUF - 1) % _NBUF)

                drain_compute(g, r)
            return carry

        lax.fori_loop(0, ngrp // _NBUF, body, 0)
        pltpu.sync_copy(out_v.at[pl.ds(0, bpw)], out_h.at[pl.ds(base, bpw)])

    return sc_dot


def kernel(user_indices, item_indices, user_table, item_table):
    sc_dot = _build(user_table.shape[0])
    return sc_dot(user_table.T, item_table.T,
                  user_indices.astype(jnp.int32),
                  item_indices.astype(jnp.int32))


# speedup vs baseline: 1.0462x; 1.0462x over previous
import functools

import jax
import jax.numpy as jnp
from jax import lax
from jax.experimental import pallas as pl
from jax.experimental.pallas import tpu as pltpu
from jax.experimental.pallas import tpu_sc as plsc

_B = 16384
_F = 32
_G = 3
_NBUF = 4


@functools.cache
def _build(num_rows):
    info = plsc.get_sparse_core_info()
    nc, ns, nl = info.num_cores, info.num_subcores, info.num_lanes
    nw = nc * ns
    bpw = _B // nw
    ngrp = (bpw + _G - 1) // _G
    ngrp = ((ngrp + _NBUF - 1) // _NBUF) * _NBUF
    pad = ngrp * _G + nl
    max_c = (num_rows + 127) // 128 - 1
    mesh = plsc.VectorSubcoreMesh(core_axis_name="c", subcore_axis_name="s")

    @functools.partial(
        pl.kernel,
        mesh=mesh,
        out_type=jax.ShapeDtypeStruct((_B,), jnp.float32),
        compiler_params=pltpu.CompilerParams(needs_layout_passes=False),
        scratch_types=[
            pltpu.VMEM((pad,), jnp.int32),
            pltpu.VMEM((pad,), jnp.int32),
            pltpu.VMEM((2 * _G, _F, 128), jnp.float32),
            pltpu.VMEM((2 * _G, _F, 128), jnp.float32),
            pltpu.VMEM((2 * _G, _F, 128), jnp.float32),
            pltpu.VMEM((2 * _G, _F, 128), jnp.float32),
            pltpu.VMEM((pad,), jnp.float32),
            pltpu.SemaphoreType.DMA,
            pltpu.SemaphoreType.DMA,
            pltpu.SemaphoreType.DMA,
            pltpu.SemaphoreType.DMA,
            pltpu.SemaphoreType.DMA,
            pltpu.SemaphoreType.DMA,
            pltpu.SemaphoreType.DMA,
            pltpu.SemaphoreType.DMA,
        ],
    )
    def sc_dot(ut_h, it_h, uidx_h, iidx_h, out_h,
               uidx_v, iidx_v, b0, b1, b2, b3, out_v,
               su0, si0, su1, si1, su2, si2, su3, si3):
        wid = lax.axis_index("s") * nc + lax.axis_index("c")
        base = wid * bpw
        pltpu.sync_copy(uidx_h.at[pl.ds(base, bpw)], uidx_v.at[pl.ds(0, bpw)])
        pltpu.sync_copy(iidx_h.at[pl.ds(base, bpw)], iidx_v.at[pl.ds(0, bpw)])

        bufs = (b0, b1, b2, b3)
        sems = ((su0, si0), (su1, si1), (su2, si2), (su3, si3))
        lanes = lax.iota(jnp.int32, nl)
        blk_u = jnp.minimum(2 * lanes, 2 * _G - 2)
        blk_i = jnp.minimum(2 * lanes + 1, 2 * _G - 1)

        def fire(g, b):
            buf, (sem_u, sem_i) = bufs[b], sems[b]
            iv_u = uidx_v[pl.ds(g * _G, nl)]
            iv_i = iidx_v[pl.ds(g * _G, nl)]
            for j in range(_G):
                c_u = jnp.clip(iv_u[j] >> 7, 0, max_c)
                c_i = jnp.clip(iv_i[j] >> 7, 0, max_c)
                off_u = pl.multiple_of(c_u * 128, 128)
                off_i = pl.multiple_of(c_i * 128, 128)
                pltpu.async_copy(ut_h.at[pl.ds(0, _F), pl.ds(off_u, 128)],
                                 buf.at[2 * j], sem_u)
                pltpu.async_copy(it_h.at[pl.ds(0, _F), pl.ds(off_i, 128)],
                                 buf.at[2 * j + 1], sem_i)

        def drain_compute(g, b):
            buf, (sem_u, sem_i) = bufs[b], sems[b]
            for j in range(_G):
                pltpu.make_async_copy(ut_h.at[pl.ds(0, _F), pl.ds(0, 128)],
                                      buf.at[2 * j], sem_u).wait()
                pltpu.make_async_copy(it_h.at[pl.ds(0, _F), pl.ds(0, 128)],
                                      buf.at[2 * j + 1], sem_i).wait()
            iv_u = uidx_v[pl.ds(g * _G, nl)]
            iv_i = iidx_v[pl.ds(g * _G, nl)]
            q_u = jnp.bitwise_and(iv_u, 127)
            q_i = jnp.bitwise_and(iv_i, 127)
            acc = jnp.zeros((nl,), jnp.float32)
            for f in range(_F):
                fv = jnp.full((nl,), f, jnp.int32)
                gu = plsc.load_gather(buf, [blk_u, fv, q_u])
                gi = plsc.load_gather(buf, [blk_i, fv, q_i])
                acc = acc + gu * gi
            out_v[pl.ds(g * _G, nl)] = acc

        for b in range(_NBUF - 1):
            fire(b, b)

        def body(p, carry):
            g0 = _NBUF * p
            for r in range(_NBUF):
                g = g0 + r

                @pl.when(g + _NBUF - 1 < ngrp)
                def _(g=g, r=r):
                    fire(g + _NBUF - 1, (r + _NBUF - 1) % _NBUF)

                drain_compute(g, r)
            return carry

        lax.fori_loop(0, ngrp // _NBUF, body, 0)
        pltpu.sync_copy(out_v.at[pl.ds(0, bpw)], out_h.at[pl.ds(base, bpw)])

    return sc_dot


def kernel(user_indices, item_indices, user_table, item_table):
    sc_dot = _build(user_table.shape[0])
    return sc_dot(user_table.T, item_table.T,
                  user_indices.astype(jnp.int32),
                  item_indices.astype(jnp.int32))
